# baseline (speedup 1.0000x reference)
import functools

import jax
import jax.numpy as jnp
from jax import lax
from jax.experimental import pallas as pl
from jax.experimental.pallas import tpu as pltpu
from jax.experimental.pallas import tpu_sc as plsc


def _make_sc_gather(n_rows, d, nc, ns, chunk):
    nw = nc * ns
    rows_per_w = n_rows // nw
    n_chunks = rows_per_w // chunk
    assert n_chunks >= 4 and n_chunks % 2 == 0
    mesh = plsc.VectorSubcoreMesh(core_axis_name="c", subcore_axis_name="s")

    @functools.partial(
        pl.kernel,
        mesh=mesh,
        out_type=jax.ShapeDtypeStruct((n_rows, d), jnp.float32),
        scratch_types=[
            pltpu.VMEM((n_chunks, chunk), jnp.int32),
            pltpu.VMEM((3, chunk, d), jnp.float32),
            pltpu.SemaphoreType.DMA,
            pltpu.SemaphoreType.DMA,
            pltpu.SemaphoreType.DMA,
            pltpu.SemaphoreType.DMA,
            pltpu.SemaphoreType.DMA,
            pltpu.SemaphoreType.DMA,
        ],
    )
    def k(pe_hbm, idx_hbm, out_hbm, idx_v, rows_v,
          gsem0, gsem1, gsem2, ssem0, ssem1, ssem2):
        gsem = (gsem0, gsem1, gsem2)
        ssem = (ssem0, ssem1, ssem2)
        wid = lax.axis_index("s") * nc + lax.axis_index("c")
        base = wid * rows_per_w
        pltpu.sync_copy(idx_hbm.at[wid], idx_v)

        def fire_gather(i, b):
            pltpu.async_copy(pe_hbm.at[idx_v.at[i]], rows_v.at[b], gsem[b])

        def wait_gather(b):
            pltpu.make_async_copy(
                pe_hbm.at[idx_v.at[0]], rows_v.at[b], gsem[b]
            ).wait()

        def fire_store(i, b):
            pltpu.async_copy(
                rows_v.at[b], out_hbm.at[pl.ds(base + i * chunk, chunk)], ssem[b]
            )

        def wait_store(b):
            pltpu.make_async_copy(
                rows_v.at[b], out_hbm.at[pl.ds(base, chunk)], ssem[b]
            ).wait()

        def step(i, b, wait_s=True, fire_g=True):
            bp = (b + 1) % 3
            if wait_s:
                wait_store(bp)
            if fire_g:
                fire_gather(i + 1, bp)
            wait_gather(b)
            fire_store(i, b)

        fire_gather(0, 0)
        step(0, 0, wait_s=False)
        step(1, 1, wait_s=False)

        def body(j0, _):
            for kk in range(3):
                step(2 + j0 * 3 + kk, (2 + kk) % 3)
            return ()

        n_loop = (n_chunks - 5) // 3
        lax.fori_loop(0, n_loop, body, ())
        tail = 2 + n_loop * 3
        for i in range(tail, n_chunks):
            step(i, i % 3, fire_g=(i + 1 < n_chunks))
        wait_store((n_chunks - 2) % 3)
        wait_store((n_chunks - 1) % 3)

    return k


def kernel(pe, frame_idx):
    b, s = frame_idx.shape
    max_len, d = pe.shape
    n_rows = b * s

    info = plsc.get_sparse_core_info()
    nc, ns = info.num_cores, info.num_subcores
    nw = nc * ns
    chunk = 32
    rows_per_w = n_rows // nw
    idx3 = frame_idx.reshape(nw, rows_per_w // chunk, chunk)

    out = _make_sc_gather(n_rows, d, nc, ns, chunk)(pe, idx3)
    return out.reshape(b, s, d)


# speedup vs baseline: 1.0007x; 1.0007x over previous
import functools

import jax
import jax.numpy as jnp
from jax import lax
from jax.experimental import pallas as pl
from jax.experimental.pallas import tpu as pltpu
from jax.experimental.pallas import tpu_sc as plsc

_NBUF = 6
_LEAD = 2
_SLAG = 3


def _make_sc_gather(n_rows, d, nc, ns, chunk):
    nw = nc * ns
    rows_per_w = n_rows // nw
    n_chunks = rows_per_w // chunk
    nbuf, lead, slag = _NBUF, _LEAD, _SLAG
    mesh = plsc.VectorSubcoreMesh(core_axis_name="c", subcore_axis_name="s")

    @functools.partial(
        pl.kernel,
        mesh=mesh,
        out_type=jax.ShapeDtypeStruct((n_rows, d), jnp.float32),
        scratch_types=[
            pltpu.VMEM((n_chunks, chunk), jnp.int32),
            pltpu.VMEM((nbuf, chunk, d), jnp.float32),
        ]
        + [pltpu.SemaphoreType.DMA] * (2 * nbuf),
    )
    def k(pe_hbm, idx_hbm, out_hbm, idx_v, rows_v, *sems):
        gsem = sems[:nbuf]
        ssem = sems[nbuf:]
        wid = lax.axis_index("s") * nc + lax.axis_index("c")
        base = wid * rows_per_w
        pltpu.sync_copy(idx_hbm.at[wid], idx_v)

        def fire_gather(i, b):
            pltpu.async_copy(pe_hbm.at[idx_v.at[i]], rows_v.at[b], gsem[b])

        def wait_gather(b):
            pltpu.make_async_copy(
                pe_hbm.at[idx_v.at[0]], rows_v.at[b], gsem[b]
            ).wait()

        def fire_store(i, b):
            pltpu.async_copy(
                rows_v.at[b], out_hbm.at[pl.ds(base + i * chunk, chunk)], ssem[b]
            )

        def wait_store(b):
            pltpu.make_async_copy(
                rows_v.at[b], out_hbm.at[pl.ds(base, chunk)], ssem[b]
            ).wait()

        def step(i, b):
            if isinstance(i, int):
                if i >= slag:
                    wait_store((b - slag) % nbuf)
                if i + lead < n_chunks:
                    fire_gather(i + lead, (b + lead) % nbuf)
            else:
                wait_store((b - slag) % nbuf)
                fire_gather(i + lead, (b + lead) % nbuf)
            wait_gather(b)
            fire_store(i, b)

        for b in range(lead):
            fire_gather(b, b)

        head = max(slag, nbuf - lead)
        tail_start = n_chunks - max(lead, slag)
        n_mid = tail_start - head
        n_loop = n_mid // nbuf
        mid_end = head + n_loop * nbuf

        for i in range(head):
            step(i, i % nbuf)

        def body(j0, _):
            for kk in range(nbuf):
                step(head + j0 * nbuf + kk, (head + kk) % nbuf)
            return ()

        lax.fori_loop(0, n_loop, body, ())

        for i in range(mid_end, n_chunks):
            step(i, i % nbuf)

        for i in range(n_chunks - slag, n_chunks):
            wait_store(i % nbuf)

    return k


def kernel(pe, frame_idx):
    b, s = frame_idx.shape
    max_len, d = pe.shape
    n_rows = b * s

    info = plsc.get_sparse_core_info()
    nc, ns = info.num_cores, info.num_subcores
    nw = nc * ns
    chunk = 16
    rows_per_w = n_rows // nw
    idx3 = frame_idx.reshape(nw, rows_per_w // chunk, chunk)

    out = _make_sc_gather(n_rows, d, nc, ns, chunk)(pe, idx3)
    return out.reshape(b, s, d)
